# Initial kernel scaffold; baseline (speedup 1.0000x reference)
import jax
import jax.numpy as jnp
from jax.experimental import pallas as pl


def kernel(x, edge_index, W1_0, b1_0, W1_1, b1_1, W1_2, b1_2, W1_3, b1_3, W2_0, b2_0, W2_1, b2_1, W2_2, b2_2, W2_3, b2_3, Wl, bl):
    raise NotImplementedError("write your pallas kernel here")


# speedup vs baseline: 7.0081x; 7.0081x over previous
import functools

import jax
import jax.numpy as jnp
from jax import lax
from jax.experimental import pallas as pl
from jax.experimental.pallas import tpu as pltpu
from jax.experimental.pallas import tpu_sc as plsc

N_NODE = 10000
N_EDGE = 320000
NC = 2
NS = 16
CH = 128
PT_CH = 160
DW = 16
N_PAD = 10016
ROW_A = 632
LAST_OFF = ROW_A * (NS - 1)
LAST_A = N_NODE - LAST_OFF
LAST_PAD = N_PAD - LAST_OFF
_PREC = jax.lax.Precision.HIGHEST

_MESH = plsc.VectorSubcoreMesh(
    core_axis_name="c", subcore_axis_name="s", num_cores=NC, num_subcores=NS
)
_SC_PARAMS = pltpu.CompilerParams(use_tc_tiling_on_sc=False)


def _acc_init(s, src, acc, nrow_src_last):
    @pl.when(s < NS - 1)
    def _():
        pltpu.sync_copy(
            src.at[pl.ds(s * ROW_A, ROW_A)], acc.at[pl.ds(s * ROW_A, ROW_A)]
        )

    @pl.when(s == NS - 1)
    def _():
        pltpu.sync_copy(
            src.at[pl.ds(LAST_OFF, nrow_src_last)],
            acc.at[pl.ds(LAST_OFF, nrow_src_last)],
        )


def _acc_out(s, acc, out, out_off):
    @pl.when(s < NS - 1)
    def _():
        pltpu.sync_copy(
            acc.at[pl.ds(s * ROW_A, ROW_A)],
            out.at[pl.ds(out_off + s * ROW_A, ROW_A)],
        )

    @pl.when(s == NS - 1)
    def _():
        pltpu.sync_copy(
            acc.at[pl.ds(LAST_OFF, LAST_A)],
            out.at[pl.ds(out_off + LAST_OFF, LAST_A)],
        )


@functools.lru_cache(maxsize=None)
def _make_deg():
    half = PT_CH // 2

    @functools.partial(
        pl.kernel,
        out_type=jax.ShapeDtypeStruct((NC * N_NODE, DW), jnp.float32),
        mesh=_MESH,
        scratch_types=[
            pltpu.VMEM((half, CH), jnp.int32),
            pltpu.VMEM((CH, DW), jnp.float32),
            pltpu.VMEM_SHARED((N_PAD, DW), jnp.float32),
        ],
        compiler_params=_SC_PARAMS,
    )
    def _deg_kernel(dstp, ones, zeros, out, dst_v, ones_v, acc):
        c = lax.axis_index("c")
        s = lax.axis_index("s")
        pltpu.sync_copy(dstp.at[s, pl.ds(c * half, half)], dst_v)
        pltpu.sync_copy(ones, ones_v)
        _acc_init(s, zeros, acc, LAST_PAD)
        plsc.subcore_barrier()

        def grp(g, carry):
            pltpu.sync_copy(ones_v, acc.at[dst_v.at[g]], add=True)
            return carry

        lax.fori_loop(0, half, grp, 0)
        plsc.subcore_barrier()
        _acc_out(s, acc, out, c * N_NODE)

    return _deg_kernel


@functools.lru_cache(maxsize=None)
def _make_pass(wc: int):
    nbuf = 2
    ngrp = PT_CH // nbuf

    @functools.partial(
        pl.kernel,
        out_type=jax.ShapeDtypeStruct((NC * N_NODE, wc), jnp.float32),
        mesh=_MESH,
        scratch_types=[
            pltpu.VMEM((PT_CH, CH), jnp.int32),
            pltpu.VMEM((PT_CH, CH), jnp.int32),
            pltpu.VMEM((nbuf, CH, wc), jnp.float32),
            pltpu.VMEM_SHARED((N_PAD, wc), jnp.float32),
        ]
        + [pltpu.SemaphoreType.DMA] * nbuf,
        compiler_params=_SC_PARAMS,
    )
    def _pass(gflat, srcp2, dstp, out, src_v, dst_v, bufs, acc, *gsem):
        c = lax.axis_index("c")
        s = lax.axis_index("s")
        pltpu.sync_copy(srcp2.at[c, s], src_v)
        pltpu.sync_copy(dstp.at[s], dst_v)
        _acc_init(s, gflat.at[pl.ds(c * N_NODE, N_NODE)], acc, LAST_A)

        @pl.when(s == NS - 1)
        def _():
            pltpu.sync_copy(
                gflat.at[pl.ds(c * N_NODE, N_PAD - N_NODE)],
                acc.at[pl.ds(N_NODE, N_PAD - N_NODE)],
            )
        plsc.subcore_barrier()

        def grp(g, carry):
            gds = []
            for b in range(nbuf):
                j = g * nbuf + b
                gds.append(
                    pltpu.async_copy(gflat.at[src_v.at[j]], bufs.at[b], gsem[b])
                )
            for b in range(nbuf):
                j = g * nbuf + b
                gds[b].wait()
                pltpu.sync_copy(bufs.at[b], acc.at[dst_v.at[j]], add=True)
            return carry

        lax.fori_loop(0, ngrp, grp, 0)
        plsc.subcore_barrier()
        _acc_out(s, acc, out, c * N_NODE)

    return _pass


_R = 1000
_REALS = {300: (80, 70), 100: (50,), 128: (64,)}
_PW = {300: 80, 100: 64, 128: 64}


def _merge_parts(s_refs, reals):
    cols = []
    for j in (0, 1):
        for p, w in zip(s_refs, reals):
            cols.append(p[j][:, :w])
    return jnp.concatenate(cols, axis=1)


def _emit_parts(g, reals, pw, out_refs):
    hc = sum(reals)
    off = 0
    for k, w in enumerate(reals):
        for j in (0, 1):
            sl = g[:, j * hc + off:j * hc + off + w]
            if w < pw:
                z = jnp.zeros((g.shape[0], pw - w), g.dtype)
                sl = jnp.concatenate([sl, z], axis=1)
            out_refs[k][j] = sl
        off += w


def _pre_call(x, degs):
    def body(x_ref, d0_ref, d1_ref, g_ref, dinv_ref):
        dinv = lax.rsqrt(d0_ref[:, 0:1] + d1_ref[:, 0:1] + 1.0)
        dinv_ref[...] = dinv
        xb = x_ref[...]
        g_ref[0] = xb[:, :64] * dinv
        g_ref[1] = xb[:, 64:] * dinv

    return pl.pallas_call(
        body,
        grid=(N_NODE // _R,),
        in_specs=[
            pl.BlockSpec((_R, 128), lambda i: (i, 0)),
            pl.BlockSpec((_R, DW), lambda i: (i, 0)),
            pl.BlockSpec((_R, DW), lambda i: (i, 0)),
        ],
        out_specs=[
            pl.BlockSpec((2, _R, 64), lambda i: (0, i, 0)),
            pl.BlockSpec((_R, 1), lambda i: (i, 0)),
        ],
        out_shape=[
            jax.ShapeDtypeStruct((2, N_NODE, 64), jnp.float32),
            jax.ShapeDtypeStruct((N_NODE, 1), jnp.float32),
        ],
    )(x, degs[:N_NODE], degs[N_NODE:])


def _stage_a_call(S0, W0, b0, W1, dinv):
    reals, pw = _REALS[300], _PW[300]
    nout = len(reals)

    def body(s_ref, w0_ref, b0_ref, w1_ref, dinv_ref, *out_refs):
        dv = dinv_ref[...]
        v = jnp.concatenate([s_ref[0] * dv, s_ref[1] * dv], axis=1)
        h = jnp.maximum(
            jnp.dot(v, w0_ref[...], precision=_PREC) + b0_ref[...], 0.0
        )
        g = jnp.dot(h, w1_ref[...], precision=_PREC) * dv
        _emit_parts(g, reals, pw, out_refs)

    return pl.pallas_call(
        body,
        grid=(N_NODE // _R,),
        in_specs=[
            pl.BlockSpec((2, _R, 64), lambda i: (0, i, 0)),
            pl.BlockSpec((128, 300), lambda i: (0, 0)),
            pl.BlockSpec((1, 300), lambda i: (0, 0)),
            pl.BlockSpec((300, 300), lambda i: (0, 0)),
            pl.BlockSpec((_R, 1), lambda i: (i, 0)),
        ],
        out_specs=[pl.BlockSpec((2, _R, pw), lambda i: (0, i, 0))] * nout,
        out_shape=[jax.ShapeDtypeStruct((2, N_NODE, pw), jnp.float32)] * nout,
    )(S0, W0, b0, W1, dinv)


def _stage_call(S_parts, b_prev, W, dinv, win, wout):
    rin, pin = _REALS[win], _PW[win]
    rout, po = _REALS[wout], _PW[wout]
    nin, nout = len(rin), len(rout)

    def body(*refs):
        s_refs = refs[:nin]
        b_ref, w_ref, dinv_ref = refs[nin], refs[nin + 1], refs[nin + 2]
        out_refs = refs[nin + 3:]
        dv = dinv_ref[...]
        u = _merge_parts(s_refs, rin)
        h = jnp.maximum(u * dv + b_ref[...], 0.0)
        g = jnp.dot(h, w_ref[...], precision=_PREC) * dv
        _emit_parts(g, rout, po, out_refs)

    return pl.pallas_call(
        body,
        grid=(N_NODE // _R,),
        in_specs=[pl.BlockSpec((2, _R, pin), lambda i: (0, i, 0))] * nin
        + [
            pl.BlockSpec((1, win), lambda i: (0, 0)),
            pl.BlockSpec((win, wout), lambda i: (0, 0)),
            pl.BlockSpec((_R, 1), lambda i: (i, 0)),
        ],
        out_specs=[pl.BlockSpec((2, _R, po), lambda i: (0, i, 0))] * nout,
        out_shape=[jax.ShapeDtypeStruct((2, N_NODE, po), jnp.float32)] * nout,
    )(*S_parts, b_prev, W, dinv)


def _final_call(S, b_prev, Wl, bl, dinv):
    def body(s_ref, b_ref, wl_ref, bl_ref, dinv_ref, out_ref):
        dv = dinv_ref[...]
        u = jnp.concatenate([s_ref[0][:, :50], s_ref[1][:, :50]], axis=1)
        h = jnp.maximum(u * dv + b_ref[...], 0.0)
        y = jnp.dot(h, wl_ref[...], precision=_PREC) + bl_ref[...]
        out_ref[...] = jax.nn.sigmoid(y)

    return pl.pallas_call(
        body,
        grid=(N_NODE // _R,),
        in_specs=[
            pl.BlockSpec((2, _R, 64), lambda i: (0, i, 0)),
            pl.BlockSpec((1, 100), lambda i: (0, 0)),
            pl.BlockSpec((100, 1), lambda i: (0, 0)),
            pl.BlockSpec((1, 1), lambda i: (0, 0)),
            pl.BlockSpec((_R, 1), lambda i: (i, 0)),
        ],
        out_specs=pl.BlockSpec((_R, 1), lambda i: (i, 0)),
        out_shape=jax.ShapeDtypeStruct((N_NODE, 1), jnp.float32),
    )(S, b_prev, Wl, bl, dinv)


def kernel(x, edge_index, W1_0, b1_0, W1_1, b1_1, W1_2, b1_2, W1_3, b1_3,
           W2_0, b2_0, W2_1, b2_1, W2_2, b2_2, W2_3, b2_3, Wl, bl):
    src = edge_index[0]
    dst = edge_index[1]
    per_tile = N_EDGE // NS
    pad = PT_CH * CH - per_tile
    srcp = jnp.concatenate(
        [src.reshape(NS, per_tile), jnp.zeros((NS, pad), jnp.int32)], axis=1
    ).reshape(NS, PT_CH, CH)
    dstp = jnp.concatenate(
        [dst.reshape(NS, per_tile), jnp.full((NS, pad), N_NODE, jnp.int32)], axis=1
    ).reshape(NS, PT_CH, CH)
    srcp2 = srcp[None] + jnp.array([0, N_NODE], jnp.int32).reshape(2, 1, 1, 1)

    onehot = jnp.zeros((CH, DW), jnp.float32).at[:, 0].set(1.0)
    zeros = jnp.zeros((N_PAD, DW), jnp.float32)
    degs = _make_deg()(dstp, onehot, zeros)

    g, dinv = _pre_call(x, degs)

    p64 = _make_pass(64)
    p80 = _make_pass(80)

    def spass(p, part, wc):
        return p(part.reshape(2 * N_NODE, wc), srcp2, dstp).reshape(2, N_NODE, wc)

    S = spass(p64, g, 64)
    gp = _stage_a_call(S, W1_0, b1_0.reshape(1, 300), W1_1, dinv)

    for b_prev, W, win, wout in [
        (b1_1, W1_2, 300, 300),
        (b1_2, W1_3, 300, 300),
        (b1_3, W2_0, 300, 100),
    ]:
        Sp = [spass(p80, part, 80) for part in gp]
        gp = _stage_call(Sp, b_prev.reshape(1, win), W, dinv, win, wout)

    for b_prev, W in [(b2_0, W2_1), (b2_1, W2_2), (b2_2, W2_3)]:
        S = spass(p64, gp[0], 64)
        gp = _stage_call([S], b_prev.reshape(1, 100), W, dinv, 100, 100)

    S = spass(p64, gp[0], 64)
    return _final_call(S, b2_3.reshape(1, 100), Wl, bl.reshape(1, 1), dinv)


# speedup vs baseline: 8.5325x; 1.2175x over previous
import functools

import jax
import jax.numpy as jnp
from jax import lax
from jax.experimental import pallas as pl
from jax.experimental.pallas import tpu as pltpu
from jax.experimental.pallas import tpu_sc as plsc

N_NODE = 10000
N_EDGE = 320000
NC = 2
NS = 16
CH = 128
PT_CH = 160
DW = 16
N_PAD = 10016
ROW_A = 632
LAST_OFF = ROW_A * (NS - 1)
LAST_A = N_NODE - LAST_OFF
LAST_PAD = N_PAD - LAST_OFF
_PREC = jax.lax.Precision.HIGHEST

_MESH = plsc.VectorSubcoreMesh(
    core_axis_name="c", subcore_axis_name="s", num_cores=NC, num_subcores=NS
)
_SC_PARAMS = pltpu.CompilerParams(use_tc_tiling_on_sc=False)


def _acc_init(s, src, acc, nrow_src_last):
    @pl.when(s < NS - 1)
    def _():
        pltpu.sync_copy(
            src.at[pl.ds(s * ROW_A, ROW_A)], acc.at[pl.ds(s * ROW_A, ROW_A)]
        )

    @pl.when(s == NS - 1)
    def _():
        pltpu.sync_copy(
            src.at[pl.ds(LAST_OFF, nrow_src_last)],
            acc.at[pl.ds(LAST_OFF, nrow_src_last)],
        )


def _acc_out(s, acc, out, out_off):
    @pl.when(s < NS - 1)
    def _():
        pltpu.sync_copy(
            acc.at[pl.ds(s * ROW_A, ROW_A)],
            out.at[pl.ds(out_off + s * ROW_A, ROW_A)],
        )

    @pl.when(s == NS - 1)
    def _():
        pltpu.sync_copy(
            acc.at[pl.ds(LAST_OFF, LAST_A)],
            out.at[pl.ds(out_off + LAST_OFF, LAST_A)],
        )


@functools.lru_cache(maxsize=None)
def _make_deg():
    half = PT_CH // 2

    @functools.partial(
        pl.kernel,
        out_type=jax.ShapeDtypeStruct((NC * N_NODE, DW), jnp.float32),
        mesh=_MESH,
        scratch_types=[
            pltpu.VMEM((half, CH), jnp.int32),
            pltpu.VMEM((CH, DW), jnp.float32),
            pltpu.VMEM_SHARED((N_PAD, DW), jnp.float32),
            pltpu.SemaphoreType.DMA,
        ],
        compiler_params=_SC_PARAMS,
    )
    def _deg_kernel(dstp, ones, zeros, out, dst_v, ones_v, acc, sem):
        c = lax.axis_index("c")
        s = lax.axis_index("s")
        pltpu.sync_copy(dstp.at[s, pl.ds(c * half, half)], dst_v)
        pltpu.sync_copy(ones, ones_v)
        _acc_init(s, zeros, acc, LAST_PAD)
        plsc.subcore_barrier()

        for g0 in range(0, half, 8):
            ds_ = [
                pltpu.async_copy(ones_v, acc.at[dst_v.at[g0 + k]], sem, add=True)
                for k in range(8)
            ]
            for d in ds_:
                d.wait()
        plsc.subcore_barrier()
        _acc_out(s, acc, out, c * N_NODE)

    return _deg_kernel


@functools.lru_cache(maxsize=None)
def _make_pass(wc: int):
    nbuf = 5 if wc <= 64 else 3
    la = 3 if wc <= 64 else 2

    @functools.partial(
        pl.kernel,
        out_type=jax.ShapeDtypeStruct((NC * N_NODE, wc), jnp.float32),
        mesh=_MESH,
        scratch_types=[
            pltpu.VMEM((PT_CH, CH), jnp.int32),
            pltpu.VMEM((PT_CH, CH), jnp.int32),
            pltpu.VMEM((nbuf, CH, wc), jnp.float32),
            pltpu.VMEM_SHARED((N_PAD, wc), jnp.float32),
        ]
        + [pltpu.SemaphoreType.DMA] * (2 * nbuf),
        compiler_params=_SC_PARAMS,
    )
    def _pass(gflat, srcp2, dstp, out, src_v, dst_v, bufs, acc, *sems):
        gsem = sems[:nbuf]
        ssem = sems[nbuf:]
        c = lax.axis_index("c")
        s = lax.axis_index("s")
        pltpu.sync_copy(srcp2.at[c, s], src_v)
        pltpu.sync_copy(dstp.at[s], dst_v)
        _acc_init(s, gflat.at[pl.ds(c * N_NODE, N_NODE)], acc, LAST_A)

        @pl.when(s == NS - 1)
        def _():
            pltpu.sync_copy(
                gflat.at[pl.ds(c * N_NODE, N_PAD - N_NODE)],
                acc.at[pl.ds(N_NODE, N_PAD - N_NODE)],
            )
        plsc.subcore_barrier()

        def gather(m):
            return pltpu.async_copy(
                gflat.at[src_v.at[m]], bufs.at[m % nbuf], gsem[m % nbuf]
            )

        gds = [None] * PT_CH
        sds = [None] * PT_CH
        for k in range(la):
            gds[k] = gather(k)
        for j in range(PT_CH):
            b = j % nbuf
            gds[j].wait()
            sds[j] = pltpu.async_copy(
                bufs.at[b], acc.at[dst_v.at[j]], ssem[b], add=True
            )
            if j + la < PT_CH:
                jj = j - (nbuf - la)
                if jj >= 0:
                    sds[jj].wait()
                gds[j + la] = gather(j + la)
        for j in range(PT_CH - nbuf, PT_CH):
            sds[j].wait()

        plsc.subcore_barrier()
        _acc_out(s, acc, out, c * N_NODE)

    return _pass


_R = 1000
_REALS = {300: (80, 70), 100: (50,), 128: (64,)}
_PW = {300: 80, 100: 64, 128: 64}


def _merge_parts(s_refs, reals):
    cols = []
    for j in (0, 1):
        for p, w in zip(s_refs, reals):
            cols.append(p[j][:, :w])
    return jnp.concatenate(cols, axis=1)


def _emit_parts(g, reals, pw, out_refs):
    hc = sum(reals)
    off = 0
    for k, w in enumerate(reals):
        for j in (0, 1):
            sl = g[:, j * hc + off:j * hc + off + w]
            if w < pw:
                z = jnp.zeros((g.shape[0], pw - w), g.dtype)
                sl = jnp.concatenate([sl, z], axis=1)
            out_refs[k][j] = sl
        off += w


def _pre_call(x, degs):
    def body(x_ref, d0_ref, d1_ref, g_ref, dinv_ref):
        dinv = lax.rsqrt(d0_ref[:, 0:1] + d1_ref[:, 0:1] + 1.0)
        dinv_ref[...] = dinv
        xb = x_ref[...]
        g_ref[0] = xb[:, :64] * dinv
        g_ref[1] = xb[:, 64:] * dinv

    return pl.pallas_call(
        body,
        grid=(N_NODE // _R,),
        in_specs=[
            pl.BlockSpec((_R, 128), lambda i: (i, 0)),
            pl.BlockSpec((_R, DW), lambda i: (i, 0)),
            pl.BlockSpec((_R, DW), lambda i: (i, 0)),
        ],
        out_specs=[
            pl.BlockSpec((2, _R, 64), lambda i: (0, i, 0)),
            pl.BlockSpec((_R, 1), lambda i: (i, 0)),
        ],
        out_shape=[
            jax.ShapeDtypeStruct((2, N_NODE, 64), jnp.float32),
            jax.ShapeDtypeStruct((N_NODE, 1), jnp.float32),
        ],
    )(x, degs[:N_NODE], degs[N_NODE:])


def _stage_a_call(S0, W0, b0, W1, dinv):
    reals, pw = _REALS[300], _PW[300]
    nout = len(reals)

    def body(s_ref, w0_ref, b0_ref, w1_ref, dinv_ref, *out_refs):
        dv = dinv_ref[...]
        v = jnp.concatenate([s_ref[0] * dv, s_ref[1] * dv], axis=1)
        h = jnp.maximum(
            jnp.dot(v, w0_ref[...], precision=_PREC) + b0_ref[...], 0.0
        )
        g = jnp.dot(h, w1_ref[...], precision=_PREC) * dv
        _emit_parts(g, reals, pw, out_refs)

    return pl.pallas_call(
        body,
        grid=(N_NODE // _R,),
        in_specs=[
            pl.BlockSpec((2, _R, 64), lambda i: (0, i, 0)),
            pl.BlockSpec((128, 300), lambda i: (0, 0)),
            pl.BlockSpec((1, 300), lambda i: (0, 0)),
            pl.BlockSpec((300, 300), lambda i: (0, 0)),
            pl.BlockSpec((_R, 1), lambda i: (i, 0)),
        ],
        out_specs=[pl.BlockSpec((2, _R, pw), lambda i: (0, i, 0))] * nout,
        out_shape=[jax.ShapeDtypeStruct((2, N_NODE, pw), jnp.float32)] * nout,
    )(S0, W0, b0, W1, dinv)


def _stage_call(S_parts, b_prev, W, dinv, win, wout):
    rin, pin = _REALS[win], _PW[win]
    rout, po = _REALS[wout], _PW[wout]
    nin, nout = len(rin), len(rout)

    def body(*refs):
        s_refs = refs[:nin]
        b_ref, w_ref, dinv_ref = refs[nin], refs[nin + 1], refs[nin + 2]
        out_refs = refs[nin + 3:]
        dv = dinv_ref[...]
        u = _merge_parts(s_refs, rin)
        h = jnp.maximum(u * dv + b_ref[...], 0.0)
        g = jnp.dot(h, w_ref[...], precision=_PREC) * dv
        _emit_parts(g, rout, po, out_refs)

    return pl.pallas_call(
        body,
        grid=(N_NODE // _R,),
        in_specs=[pl.BlockSpec((2, _R, pin), lambda i: (0, i, 0))] * nin
        + [
            pl.BlockSpec((1, win), lambda i: (0, 0)),
            pl.BlockSpec((win, wout), lambda i: (0, 0)),
            pl.BlockSpec((_R, 1), lambda i: (i, 0)),
        ],
        out_specs=[pl.BlockSpec((2, _R, po), lambda i: (0, i, 0))] * nout,
        out_shape=[jax.ShapeDtypeStruct((2, N_NODE, po), jnp.float32)] * nout,
    )(*S_parts, b_prev, W, dinv)


def _final_call(S, b_prev, Wl, bl, dinv):
    def body(s_ref, b_ref, wl_ref, bl_ref, dinv_ref, out_ref):
        dv = dinv_ref[...]
        u = jnp.concatenate([s_ref[0][:, :50], s_ref[1][:, :50]], axis=1)
        h = jnp.maximum(u * dv + b_ref[...], 0.0)
        y = jnp.dot(h, wl_ref[...], precision=_PREC) + bl_ref[...]
        out_ref[...] = jax.nn.sigmoid(y)

    return pl.pallas_call(
        body,
        grid=(N_NODE // _R,),
        in_specs=[
            pl.BlockSpec((2, _R, 64), lambda i: (0, i, 0)),
            pl.BlockSpec((1, 100), lambda i: (0, 0)),
            pl.BlockSpec((100, 1), lambda i: (0, 0)),
            pl.BlockSpec((1, 1), lambda i: (0, 0)),
            pl.BlockSpec((_R, 1), lambda i: (i, 0)),
        ],
        out_specs=pl.BlockSpec((_R, 1), lambda i: (i, 0)),
        out_shape=jax.ShapeDtypeStruct((N_NODE, 1), jnp.float32),
    )(S, b_prev, Wl, bl, dinv)


def kernel(x, edge_index, W1_0, b1_0, W1_1, b1_1, W1_2, b1_2, W1_3, b1_3,
           W2_0, b2_0, W2_1, b2_1, W2_2, b2_2, W2_3, b2_3, Wl, bl):
    src = edge_index[0]
    dst = edge_index[1]
    per_tile = N_EDGE // NS
    pad = PT_CH * CH - per_tile
    srcp = jnp.concatenate(
        [src.reshape(NS, per_tile), jnp.zeros((NS, pad), jnp.int32)], axis=1
    ).reshape(NS, PT_CH, CH)
    dstp = jnp.concatenate(
        [dst.reshape(NS, per_tile), jnp.full((NS, pad), N_NODE, jnp.int32)], axis=1
    ).reshape(NS, PT_CH, CH)
    srcp2 = srcp[None] + jnp.array([0, N_NODE], jnp.int32).reshape(2, 1, 1, 1)

    onehot = jnp.zeros((CH, DW), jnp.float32).at[:, 0].set(1.0)
    zeros = jnp.zeros((N_PAD, DW), jnp.float32)
    degs = _make_deg()(dstp, onehot, zeros)

    g, dinv = _pre_call(x, degs)

    p64 = _make_pass(64)
    p80 = _make_pass(80)

    def spass(p, part, wc):
        return p(part.reshape(2 * N_NODE, wc), srcp2, dstp).reshape(2, N_NODE, wc)

    S = spass(p64, g, 64)
    gp = _stage_a_call(S, W1_0, b1_0.reshape(1, 300), W1_1, dinv)

    for b_prev, W, win, wout in [
        (b1_1, W1_2, 300, 300),
        (b1_2, W1_3, 300, 300),
        (b1_3, W2_0, 300, 100),
    ]:
        Sp = [spass(p80, part, 80) for part in gp]
        gp = _stage_call(Sp, b_prev.reshape(1, win), W, dinv, win, wout)

    for b_prev, W in [(b2_0, W2_1), (b2_1, W2_2), (b2_2, W2_3)]:
        S = spass(p64, gp[0], 64)
        gp = _stage_call([S], b_prev.reshape(1, 100), W, dinv, 100, 100)

    S = spass(p64, gp[0], 64)
    return _final_call(S, b2_3.reshape(1, 100), Wl, bl.reshape(1, 1), dinv)
